# baseline (device time: 265835 ns/iter reference)
import jax
import jax.numpy as jnp
from jax import lax
from jax.experimental import pallas as pl
from jax.experimental.pallas import tpu as pltpu

N_DEV = 8
B_LOC = 2
SQ = 512
D = 1024
H_LOC = 8
DH = 128
ROWS = B_LOC * SQ
SCALE = 0.08838834764831843
F32 = jnp.float32
BF16 = jnp.bfloat16


def _body(x_ref, wq_ref, wk_ref, wv_ref, wo_ref, cos_ref, sin_ref,
          out_ref, xfull, xown, landr, landl, ctx_sc,
          agr_s, agr_r, agl_s, agl_r, rsr_s, rsr_r, rsl_s, rsl_r):
    me = lax.axis_index("i")
    right = lax.rem(me + 1, N_DEV)
    left = lax.rem(me + N_DEV - 1, N_DEV)

    def rcopy(src, dst, ssem, rsem, dev):
        return pltpu.make_async_remote_copy(
            src_ref=src, dst_ref=dst, send_sem=ssem, recv_sem=rsem,
            device_id=(dev,), device_id_type=pl.DeviceIdType.MESH)

    def agR(h):
        src = xown if h == 0 else xfull.at[7 - h]
        return rcopy(src, xfull.at[6 - h], agr_s.at[h], agr_r.at[h], right)

    def agL(g):
        src = xown if g == 0 else xfull.at[g - 1]
        return rcopy(src, xfull.at[g], agl_s.at[g], agl_r.at[g], left)

    def rsR(t):
        src = xfull.at[2] if t == 0 else landr.at[t - 1]
        return rcopy(src, landr.at[t], rsr_s.at[t], rsr_r.at[t], right)

    def rsL(t):
        src = xfull.at[3] if t == 0 else landl.at[t - 1]
        return rcopy(src, landl.at[t], rsl_s.at[t], rsl_r.at[t], left)

    barrier = pltpu.get_barrier_semaphore()
    for nbr in (left, right):
        pl.semaphore_signal(barrier, inc=1, device_id=(nbr,),
                            device_id_type=pl.DeviceIdType.MESH)
    pl.semaphore_wait(barrier, 2)

    xown[...] = x_ref[...].reshape(ROWS, D)
    agR(0).start()
    agL(0).start()

    cos = cos_ref[...]
    sin = sin_ref[...]
    ones_col = jnp.ones((SQ, 1), BF16)

    def rope(t, c, s):
        tr = jnp.concatenate([-t[:, DH // 2:], t[:, :DH // 2]], axis=1)
        return (t * c + tr * s).astype(BF16)

    def compute(xsrc):
        for bb in range(B_LOC):
            r0 = bb * SQ
            xb = xsrc[r0:r0 + SQ, :]
            q = jnp.dot(xb, wq_ref[...], preferred_element_type=F32)
            k = jnp.dot(xb, wk_ref[...], preferred_element_type=F32)
            v = jnp.dot(xb, wv_ref[...],
                        preferred_element_type=F32).astype(BF16)
            for h in range(H_LOC):
                c0 = h * DH
                qh = rope(q[:, c0:c0 + DH], cos[:, :DH], sin[:, :DH])
                kh = rope(k[:, c0:c0 + DH], cos[:, DH:], sin[:, DH:])
                s = lax.dot_general(
                    qh, kh, (((1,), (1,)), ((), ())),
                    preferred_element_type=F32)
                w = jnp.exp(s).astype(BF16)
                denom = lax.dot_general(
                    w, ones_col, (((1,), (0,)), ((), ())),
                    preferred_element_type=F32)
                ctx = jnp.dot(w, v[:, c0:c0 + DH],
                              preferred_element_type=F32)
                ctx_sc[r0:r0 + SQ, c0:c0 + DH] = (ctx / denom).astype(BF16)
        return jnp.dot(ctx_sc[...], wo_ref[...], preferred_element_type=F32)

    def accum(land_ref, slot_idx, part_f32):
        land_ref[slot_idx] = (land_ref[slot_idx].astype(F32)
                              + part_f32).astype(BF16)

    out_ref[...] = compute(xown).reshape(B_LOC, SQ, D)

    agR(0).wait_recv()
    agR(1).start()
    agL(0).wait_recv()
    agL(1).start()

    p_m1 = compute(xfull.at[6])
    agR(1).wait_send()
    xfull[6] = p_m1.astype(BF16)

    agR(1).wait_recv()
    agR(2).start()
    agL(1).wait_recv()
    agL(2).start()

    p_p1 = compute(xfull.at[0])
    agL(1).wait_send()
    xfull[0] = p_p1.astype(BF16)

    agL(2).wait_recv()
    agR(2).wait_recv()
    agR(3).start()

    p_p3 = compute(xfull.at[2])
    xfull[2] = p_p3.astype(BF16)
    rsR(0).start()

    agR(3).wait_recv()
    p_m4 = compute(xfull.at[3])
    xfull[3] = p_m4.astype(BF16)
    rsL(0).start()

    p_p2 = compute(xfull.at[1])
    rsR(0).wait_recv()
    accum(landr, 0, p_p2)
    rsR(1).start()

    p_m3 = compute(xfull.at[4])
    rsL(0).wait_recv()
    accum(landl, 0, p_m3)
    rsL(1).start()

    rsR(1).wait_recv()
    accum(landr, 1, xfull[0].astype(F32))
    rsR(2).start()

    p_m2 = compute(xfull.at[5])
    agR(2).wait_send()
    xfull[5] = p_m2.astype(BF16)
    rsL(1).wait_recv()
    accum(landl, 1, p_m2)
    rsL(2).start()

    rsL(2).wait_recv()
    accum(landl, 2, xfull[6].astype(F32))
    rsL(3).start()

    rsR(2).wait_recv()
    rsL(3).wait_recv()
    out_ref[...] = out_ref[...] + (
        landr[2].astype(F32) + landl[3].astype(F32)
    ).reshape(B_LOC, SQ, D)

    agR(0).wait_send()
    agL(0).wait_send()
    agR(3).wait_send()
    agL(2).wait_send()
    for t in range(3):
        rsR(t).wait_send()
    for t in range(4):
        rsL(t).wait_send()


def kernel(x, Wq, Wk, Wv, Wo):
    def perm(w):
        return (w.reshape(D, H_LOC, DH // 2, 2)
                 .transpose(0, 1, 3, 2)
                 .reshape(D, H_LOC * DH))

    wq = perm(Wq).astype(BF16)
    wk = perm(Wk).astype(BF16)
    wv = Wv.astype(BF16)
    wo = Wo.astype(BF16)
    xb = x.astype(BF16)

    inv = 1.0 / (10000.0 ** (jnp.arange(0, DH, 2, dtype=F32) / DH))
    pos = jnp.arange(SQ, dtype=F32)[:, None] * inv[None, :]
    c1 = jnp.concatenate([jnp.cos(pos), jnp.cos(pos)], axis=-1)
    s1 = jnp.concatenate([jnp.sin(pos), jnp.sin(pos)], axis=-1)
    cos = jnp.concatenate([c1 * SCALE, c1], axis=-1)
    sin = jnp.concatenate([s1 * SCALE, s1], axis=-1)

    return pl.pallas_call(
        _body,
        out_shape=jax.ShapeDtypeStruct((B_LOC, SQ, D), F32),
        in_specs=[pl.BlockSpec(memory_space=pltpu.VMEM)] * 7,
        out_specs=pl.BlockSpec(memory_space=pltpu.VMEM),
        scratch_shapes=[
            pltpu.VMEM((7, ROWS, D), BF16),
            pltpu.VMEM((ROWS, D), BF16),
            pltpu.VMEM((3, ROWS, D), BF16),
            pltpu.VMEM((4, ROWS, D), BF16),
            pltpu.VMEM((ROWS, D), BF16),
            pltpu.SemaphoreType.DMA((4,)),
            pltpu.SemaphoreType.DMA((4,)),
            pltpu.SemaphoreType.DMA((3,)),
            pltpu.SemaphoreType.DMA((3,)),
            pltpu.SemaphoreType.DMA((3,)),
            pltpu.SemaphoreType.DMA((3,)),
            pltpu.SemaphoreType.DMA((4,)),
            pltpu.SemaphoreType.DMA((4,)),
        ],
        compiler_params=pltpu.CompilerParams(
            collective_id=0,
            vmem_limit_bytes=100 * 1024 * 1024,
        ),
    )(xb, wq, wk, wv, wo, cos, sin)


# device time: 254269 ns/iter; 1.0455x vs baseline; 1.0455x over previous
import jax
import jax.numpy as jnp
from jax import lax
from jax.experimental import pallas as pl
from jax.experimental.pallas import tpu as pltpu

N_DEV = 8
B_LOC = 2
SQ = 512
D = 1024
H_LOC = 8
DH = 128
ROWS = B_LOC * SQ
SCALE = 0.08838834764831843
F32 = jnp.float32
BF16 = jnp.bfloat16


def _body(x_ref, wq_ref, wk_ref, wv_ref, wo_ref, cos_ref, sin_ref,
          out_ref, xfull, xown, landr, landl, ctx_sc,
          agr_s, agr_r, agl_s, agl_r, rsr_s, rsr_r, rsl_s, rsl_r):
    me = lax.axis_index("i")
    right = lax.rem(me + 1, N_DEV)
    left = lax.rem(me + N_DEV - 1, N_DEV)

    def rcopy(src, dst, ssem, rsem, dev):
        return pltpu.make_async_remote_copy(
            src_ref=src, dst_ref=dst, send_sem=ssem, recv_sem=rsem,
            device_id=(dev,), device_id_type=pl.DeviceIdType.MESH)

    def agR(h):
        src = xown if h == 0 else xfull.at[7 - h]
        return rcopy(src, xfull.at[6 - h], agr_s.at[h], agr_r.at[h], right)

    def agL(g):
        src = xown if g == 0 else xfull.at[g - 1]
        return rcopy(src, xfull.at[g], agl_s.at[g], agl_r.at[g], left)

    def rsR(t):
        src = xfull.at[2] if t == 0 else landr.at[t - 1]
        return rcopy(src, landr.at[t], rsr_s.at[t], rsr_r.at[t], right)

    def rsL(t):
        src = xfull.at[3] if t == 0 else landl.at[t - 1]
        return rcopy(src, landl.at[t], rsl_s.at[t], rsl_r.at[t], left)

    barrier = pltpu.get_barrier_semaphore()
    for nbr in (left, right):
        pl.semaphore_signal(barrier, inc=1, device_id=(nbr,),
                            device_id_type=pl.DeviceIdType.MESH)
    pl.semaphore_wait(barrier, 2)

    xown[...] = x_ref[...].reshape(ROWS, D)
    agR(0).start()
    agL(0).start()

    cos = cos_ref[...]
    sin = sin_ref[...]

    def rope(t, c, s):
        tr = jnp.concatenate([-t[:, DH // 2:], t[:, :DH // 2]], axis=1)
        return (t * c + tr * s).astype(BF16)

    def compute(xsrc):
        for bb in range(B_LOC):
            r0 = bb * SQ
            xb = xsrc[r0:r0 + SQ, :]
            q = jnp.dot(xb, wq_ref[...], preferred_element_type=F32)
            k = jnp.dot(xb, wk_ref[...], preferred_element_type=F32)
            v = jnp.dot(xb, wv_ref[...],
                        preferred_element_type=F32).astype(BF16)
            for h in range(H_LOC):
                c0 = h * DH
                qh = rope(q[:, c0:c0 + DH], cos[:, :DH], sin[:, :DH])
                kh = rope(k[:, c0:c0 + DH], cos[:, DH:], sin[:, DH:])
                s = lax.dot_general(
                    qh, kh, (((1,), (1,)), ((), ())),
                    preferred_element_type=F32)
                w = jnp.exp(s)
                denom = jnp.sum(w, axis=-1, keepdims=True)
                ctx = jnp.dot(w.astype(BF16), v[:, c0:c0 + DH],
                              preferred_element_type=F32)
                ctx_sc[r0:r0 + SQ, c0:c0 + DH] = (ctx / denom).astype(BF16)
        return jnp.dot(ctx_sc[...], wo_ref[...], preferred_element_type=F32)

    def accum(land_ref, slot_idx, part_f32):
        land_ref[slot_idx] = (land_ref[slot_idx].astype(F32)
                              + part_f32).astype(BF16)


    out_ref[...] = compute(xown).reshape(B_LOC, SQ, D)

    agR(0).wait_recv()
    agR(1).start()
    agL(0).wait_recv()
    agL(1).start()

    p_p1 = compute(xfull.at[0])
    agL(1).wait_send()
    xfull[0] = p_p1.astype(BF16)

    agL(1).wait_recv()
    agL(2).start()
    agR(1).wait_recv()
    agR(2).start()

    p_p2 = compute(xfull.at[1])
    agL(2).wait_send()
    xfull[1] = p_p2.astype(BF16)

    agL(2).wait_recv()
    agR(2).wait_recv()
    agR(3).start()

    p_p3 = compute(xfull.at[2])
    xfull[2] = p_p3.astype(BF16)
    rsR(0).start()

    agR(3).wait_recv()
    p_m4 = compute(xfull.at[3])
    xfull[3] = p_m4.astype(BF16)
    rsL(0).start()

    rsR(0).wait_recv()
    accum(landr, 0, xfull[1].astype(F32))
    rsR(1).start()

    p_m3 = compute(xfull.at[4])
    rsL(0).wait_recv()
    accum(landl, 0, p_m3)
    rsL(1).start()

    rsR(1).wait_recv()
    accum(landr, 1, xfull[0].astype(F32))
    rsR(2).start()

    p_m2 = compute(xfull.at[5])
    rsL(1).wait_recv()
    accum(landl, 1, p_m2)
    rsL(2).start()

    p_m1 = compute(xfull.at[6])
    rsL(2).wait_recv()
    accum(landl, 2, p_m1)
    rsL(3).start()

    rsR(2).wait_recv()
    rsL(3).wait_recv()
    out_ref[...] = out_ref[...] + (
        landr[2].astype(F32) + landl[3].astype(F32)
    ).reshape(B_LOC, SQ, D)

    agR(0).wait_send()
    agR(1).wait_send()
    agR(2).wait_send()
    agR(3).wait_send()
    agL(0).wait_send()
    for t in range(3):
        rsR(t).wait_send()
    for t in range(4):
        rsL(t).wait_send()


def kernel(x, Wq, Wk, Wv, Wo):
    def perm(w):
        return (w.reshape(D, H_LOC, DH // 2, 2)
                 .transpose(0, 1, 3, 2)
                 .reshape(D, H_LOC * DH))

    wq = perm(Wq).astype(BF16)
    wk = perm(Wk).astype(BF16)
    wv = Wv.astype(BF16)
    wo = Wo.astype(BF16)
    xb = x.astype(BF16)

    inv = 1.0 / (10000.0 ** (jnp.arange(0, DH, 2, dtype=F32) / DH))
    pos = jnp.arange(SQ, dtype=F32)[:, None] * inv[None, :]
    c1 = jnp.concatenate([jnp.cos(pos), jnp.cos(pos)], axis=-1)
    s1 = jnp.concatenate([jnp.sin(pos), jnp.sin(pos)], axis=-1)
    cos = jnp.concatenate([c1 * SCALE, c1], axis=-1)
    sin = jnp.concatenate([s1 * SCALE, s1], axis=-1)

    return pl.pallas_call(
        _body,
        out_shape=jax.ShapeDtypeStruct((B_LOC, SQ, D), F32),
        in_specs=[pl.BlockSpec(memory_space=pltpu.VMEM)] * 7,
        out_specs=pl.BlockSpec(memory_space=pltpu.VMEM),
        scratch_shapes=[
            pltpu.VMEM((7, ROWS, D), BF16),
            pltpu.VMEM((ROWS, D), BF16),
            pltpu.VMEM((3, ROWS, D), BF16),
            pltpu.VMEM((4, ROWS, D), BF16),
            pltpu.VMEM((ROWS, D), BF16),
            pltpu.SemaphoreType.DMA((4,)),
            pltpu.SemaphoreType.DMA((4,)),
            pltpu.SemaphoreType.DMA((3,)),
            pltpu.SemaphoreType.DMA((3,)),
            pltpu.SemaphoreType.DMA((3,)),
            pltpu.SemaphoreType.DMA((3,)),
            pltpu.SemaphoreType.DMA((4,)),
            pltpu.SemaphoreType.DMA((4,)),
        ],
        compiler_params=pltpu.CompilerParams(
            collective_id=0,
            vmem_limit_bytes=100 * 1024 * 1024,
        ),
    )(xb, wq, wk, wv, wo, cos, sin)


# device time: 252771 ns/iter; 1.0517x vs baseline; 1.0059x over previous
import jax
import jax.numpy as jnp
from jax import lax
from jax.experimental import pallas as pl
from jax.experimental.pallas import tpu as pltpu

N_DEV = 8
B_LOC = 2
SQ = 512
D = 1024
H_LOC = 8
DH = 128
ROWS = B_LOC * SQ
SCALE = 0.08838834764831843
F32 = jnp.float32
BF16 = jnp.bfloat16


def _body(x_ref, wqkv_ref, wo_ref, cos_ref, sin_ref,
          out_ref, xfull, xown, landr, landl, ctx_sc,
          agr_s, agr_r, agl_s, agl_r, rsr_s, rsr_r, rsl_s, rsl_r):
    me = lax.axis_index("i")
    right = lax.rem(me + 1, N_DEV)
    left = lax.rem(me + N_DEV - 1, N_DEV)

    def rcopy(src, dst, ssem, rsem, dev):
        return pltpu.make_async_remote_copy(
            src_ref=src, dst_ref=dst, send_sem=ssem, recv_sem=rsem,
            device_id=(dev,), device_id_type=pl.DeviceIdType.MESH)

    def agR(h):
        src = xown if h == 0 else xfull.at[7 - h]
        return rcopy(src, xfull.at[6 - h], agr_s.at[h], agr_r.at[h], right)

    def agL(g):
        src = xown if g == 0 else xfull.at[g - 1]
        return rcopy(src, xfull.at[g], agl_s.at[g], agl_r.at[g], left)

    def rsR(t):
        src = xfull.at[2] if t == 0 else landr.at[t - 1]
        return rcopy(src, landr.at[t], rsr_s.at[t], rsr_r.at[t], right)

    def rsL(t):
        src = xfull.at[3] if t == 0 else landl.at[t - 1]
        return rcopy(src, landl.at[t], rsl_s.at[t], rsl_r.at[t], left)

    barrier = pltpu.get_barrier_semaphore()
    for nbr in (left, right):
        pl.semaphore_signal(barrier, inc=1, device_id=(nbr,),
                            device_id_type=pl.DeviceIdType.MESH)
    pl.semaphore_wait(barrier, 2)

    xown[...] = x_ref[...].reshape(ROWS, D)
    agR(0).start()
    agL(0).start()

    cos = cos_ref[...]
    sin = sin_ref[...]

    def rope(t, c, s):
        tr = jnp.concatenate([-t[:, DH // 2:], t[:, :DH // 2]], axis=1)
        return (t * c + tr * s).astype(BF16)

    def compute(xsrc):
        for bb in range(B_LOC):
            r0 = bb * SQ
            xb = xsrc[r0:r0 + SQ, :]
            qkv = jnp.dot(xb, wqkv_ref[...], preferred_element_type=F32)
            q = qkv[:, :D]
            k = qkv[:, D:2 * D]
            v = qkv[:, 2 * D:].astype(BF16)
            for h in range(H_LOC):
                c0 = h * DH
                qh = rope(q[:, c0:c0 + DH], cos[:, :DH], sin[:, :DH])
                kh = rope(k[:, c0:c0 + DH], cos[:, DH:], sin[:, DH:])
                s = lax.dot_general(
                    qh, kh, (((1,), (1,)), ((), ())),
                    preferred_element_type=F32)
                w = jnp.exp(s)
                denom = jnp.sum(w, axis=-1, keepdims=True)
                ctx = jnp.dot(w.astype(BF16), v[:, c0:c0 + DH],
                              preferred_element_type=F32)
                ctx_sc[r0:r0 + SQ, c0:c0 + DH] = (ctx / denom).astype(BF16)
        return jnp.dot(ctx_sc[...], wo_ref[...], preferred_element_type=F32)

    def accum(land_ref, slot_idx, part_f32):
        land_ref[slot_idx] = (land_ref[slot_idx].astype(F32)
                              + part_f32).astype(BF16)


    out_ref[...] = compute(xown).reshape(B_LOC, SQ, D)

    agR(0).wait_recv()
    agR(1).start()
    agL(0).wait_recv()
    agL(1).start()

    p_p1 = compute(xfull.at[0])
    agL(1).wait_send()
    xfull[0] = p_p1.astype(BF16)

    agL(1).wait_recv()
    agL(2).start()
    agR(1).wait_recv()
    agR(2).start()

    p_p2 = compute(xfull.at[1])
    agL(2).wait_send()
    xfull[1] = p_p2.astype(BF16)

    agL(2).wait_recv()
    agR(2).wait_recv()
    agR(3).start()

    p_p3 = compute(xfull.at[2])
    xfull[2] = p_p3.astype(BF16)
    rsR(0).start()

    agR(3).wait_recv()
    p_m4 = compute(xfull.at[3])
    xfull[3] = p_m4.astype(BF16)
    rsL(0).start()

    rsR(0).wait_recv()
    accum(landr, 0, xfull[1].astype(F32))
    rsR(1).start()

    p_m3 = compute(xfull.at[4])
    rsL(0).wait_recv()
    accum(landl, 0, p_m3)
    rsL(1).start()

    rsR(1).wait_recv()
    accum(landr, 1, xfull[0].astype(F32))
    rsR(2).start()

    p_m2 = compute(xfull.at[5])
    rsL(1).wait_recv()
    accum(landl, 1, p_m2)
    rsL(2).start()

    p_m1 = compute(xfull.at[6])
    rsL(2).wait_recv()
    accum(landl, 2, p_m1)
    rsL(3).start()

    rsR(2).wait_recv()
    rsL(3).wait_recv()
    out_ref[...] = out_ref[...] + (
        landr[2].astype(F32) + landl[3].astype(F32)
    ).reshape(B_LOC, SQ, D)

    agR(0).wait_send()
    agR(1).wait_send()
    agR(2).wait_send()
    agR(3).wait_send()
    agL(0).wait_send()
    for t in range(3):
        rsR(t).wait_send()
    for t in range(4):
        rsL(t).wait_send()


def kernel(x, Wq, Wk, Wv, Wo):
    def perm(w):
        return (w.reshape(D, H_LOC, DH // 2, 2)
                 .transpose(0, 1, 3, 2)
                 .reshape(D, H_LOC * DH))

    wqkv = jnp.concatenate(
        [perm(Wq), perm(Wk), Wv], axis=1).astype(BF16)
    wo = Wo.astype(BF16)
    xb = x.astype(BF16)

    inv = 1.0 / (10000.0 ** (jnp.arange(0, DH, 2, dtype=F32) / DH))
    pos = jnp.arange(SQ, dtype=F32)[:, None] * inv[None, :]
    c1 = jnp.concatenate([jnp.cos(pos), jnp.cos(pos)], axis=-1)
    s1 = jnp.concatenate([jnp.sin(pos), jnp.sin(pos)], axis=-1)
    cos = jnp.concatenate([c1 * SCALE, c1], axis=-1)
    sin = jnp.concatenate([s1 * SCALE, s1], axis=-1)

    return pl.pallas_call(
        _body,
        out_shape=jax.ShapeDtypeStruct((B_LOC, SQ, D), F32),
        in_specs=[pl.BlockSpec(memory_space=pltpu.VMEM)] * 5,
        out_specs=pl.BlockSpec(memory_space=pltpu.VMEM),
        scratch_shapes=[
            pltpu.VMEM((7, ROWS, D), BF16),
            pltpu.VMEM((ROWS, D), BF16),
            pltpu.VMEM((3, ROWS, D), BF16),
            pltpu.VMEM((4, ROWS, D), BF16),
            pltpu.VMEM((ROWS, D), BF16),
            pltpu.SemaphoreType.DMA((4,)),
            pltpu.SemaphoreType.DMA((4,)),
            pltpu.SemaphoreType.DMA((3,)),
            pltpu.SemaphoreType.DMA((3,)),
            pltpu.SemaphoreType.DMA((3,)),
            pltpu.SemaphoreType.DMA((3,)),
            pltpu.SemaphoreType.DMA((4,)),
            pltpu.SemaphoreType.DMA((4,)),
        ],
        compiler_params=pltpu.CompilerParams(
            collective_id=0,
            vmem_limit_bytes=100 * 1024 * 1024,
        ),
    )(xb, wqkv, wo, cos, sin)
